# baseline (speedup 1.0000x reference)
import jax
import jax.numpy as jnp
import numpy as np
from jax.experimental import pallas as pl

_DIM = 256
_NH = 8
_M = 128
_TOPK = 8
_DPOS = 3
_N = 4096
_NB = _N // _M
_EH = _DIM // _NH
_NEG = np.float32(-1e5)
_SCALE = np.float32(1.0 / np.sqrt(_EH))
_TS = 512
_BQ = 256
_AUG = _EH + _NB
_bf = jnp.bfloat16
_f32 = jnp.float32


def _dot_t(a, b):
    return jax.lax.dot_general(a, b, (((1,), (1,)), ((), ())),
                               preferred_element_type=_f32)


def _qprep_kernel(x_ref, pos_ref, wpet_ref, bpe_ref, wq_ref, bq_ref,
                  wk_ref, bk_ref, q_ref, kbar_ref, kb_ref):
    i = pl.program_id(0)
    p = pos_ref[...]
    rel = (p - jnp.mean(p, axis=0, keepdims=True)).astype(_bf)
    pe = rel[:, 0:1].astype(_f32) * wpet_ref[0:1, :].astype(_f32)
    for d in range(1, _DPOS):
        pe = pe + rel[:, d:d + 1].astype(_f32) * wpet_ref[d:d + 1, :].astype(_f32)
    xp = (x_ref[...] + pe) + bpe_ref[...]
    xb = xp.astype(_bf)
    ind = (jax.lax.broadcasted_iota(jnp.int32, (_M, _NB), 1) == i).astype(_bf)
    for h in range(_NH):
        q = _dot_t(xb, wq_ref[h]) + bq_ref[h]
        q_ref[h, :, :] = q
        kb_ref[h, :, :] = jnp.concatenate([q.astype(_bf), ind], axis=1)
    k = _dot_t(xb, wk_ref[...]) + bk_ref[...]
    kbar_ref[0, :, :] = jnp.mean(k, axis=0, keepdims=True)


def _select_kernel(c_ref, q_ref, bias_ref):
    centers = c_ref[0].astype(_bf)
    q = q_ref[0].astype(_bf)
    v = _dot_t(centers, q)
    iota = jax.lax.broadcasted_iota(jnp.int32, (_NB, _TS), 0)
    sel = jnp.zeros((_NB, _TS), _f32)
    for _ in range(_TOPK):
        m = jnp.max(v, axis=0, keepdims=True)
        elig = v == m
        cand = jnp.where(elig, iota, np.int32(_NB * 2))
        bmin = jnp.min(cand, axis=0, keepdims=True)
        onehot = cand == bmin
        v = jnp.where(onehot, np.float32(-np.inf), v)
        sel = sel + onehot.astype(_f32)
    bias_ref[0, :, :] = jnp.where(sel > 0, np.float32(0.0), _NEG).astype(_bf)


def _attn_kernel(q_ref, bias_ref, kb_ref, wp_ref, bp_ref, o_ref):
    h = pl.program_id(1)
    qa = jnp.concatenate(
        [(q_ref[0] * _SCALE).astype(_bf), bias_ref[0]], axis=1)
    kb = kb_ref[h]
    s = _dot_t(qa, kb)
    p = jnp.exp(s).astype(_bf)
    o = jax.lax.dot_general(
        p, kb, (((1,), (0,)), ((), ())), preferred_element_type=_f32)
    l = jnp.sum(o[:, _EH:], axis=1, keepdims=True)
    ah = (o[:, :_EH] / l).astype(_bf)
    part = jax.lax.dot_general(
        ah, wp_ref[h], (((1,), (0,)), ((), ())), preferred_element_type=_f32)

    @pl.when(h == 0)
    def _init():
        o_ref[...] = part + bp_ref[...]

    @pl.when(h > 0)
    def _acc():
        o_ref[...] = o_ref[...] + part


def kernel(x, pos, W_qkv, b_qkv, W_proj, b_proj, W_pe, b_pe):
    Wq = W_qkv[0::3].reshape(_NH, _EH, _DIM).astype(_bf)
    bq = b_qkv[0::3].reshape(_NH, 1, _EH)
    Wk = W_qkv[1::3].astype(_bf)
    bk = b_qkv[1::3].reshape(1, _DIM)
    WpeT = W_pe.T.astype(_bf)
    bpe = b_pe.reshape(1, _DIM)
    Wp = W_proj.T.reshape(_NH, _EH, _DIM).astype(_bf)
    bp = b_proj.reshape(1, _DIM)

    qh, kbar, kb = pl.pallas_call(
        _qprep_kernel,
        grid=(_NB,),
        in_specs=[
            pl.BlockSpec((_M, _DIM), lambda i: (i, 0)),
            pl.BlockSpec((_M, _DPOS), lambda i: (i, 0)),
            pl.BlockSpec((_DPOS, _DIM), lambda i: (0, 0)),
            pl.BlockSpec((1, _DIM), lambda i: (0, 0)),
            pl.BlockSpec((_NH, _EH, _DIM), lambda i: (0, 0, 0)),
            pl.BlockSpec((_NH, 1, _EH), lambda i: (0, 0, 0)),
            pl.BlockSpec((_DIM, _DIM), lambda i: (0, 0)),
            pl.BlockSpec((1, _DIM), lambda i: (0, 0)),
        ],
        out_specs=[
            pl.BlockSpec((_NH, _M, _EH), lambda i: (0, i, 0)),
            pl.BlockSpec((1, 1, _DIM), lambda i: (i, 0, 0)),
            pl.BlockSpec((_NH, _M, _AUG), lambda i: (0, i, 0)),
        ],
        out_shape=[
            jax.ShapeDtypeStruct((_NH, _N, _EH), _f32),
            jax.ShapeDtypeStruct((_NB, 1, _DIM), _f32),
            jax.ShapeDtypeStruct((_NH, _N, _AUG), _bf),
        ],
    )(x, pos, WpeT, bpe, Wq, bq, Wk, bk)

    centers = jnp.transpose(kbar.reshape(_NB, _NH, _EH), (1, 0, 2))

    bias_bm = pl.pallas_call(
        _select_kernel,
        grid=(_NH, _N // _TS),
        in_specs=[
            pl.BlockSpec((1, _NB, _EH), lambda h, c: (h, 0, 0)),
            pl.BlockSpec((1, _TS, _EH), lambda h, c: (h, c, 0)),
        ],
        out_specs=pl.BlockSpec((1, _NB, _TS), lambda h, c: (h, 0, c)),
        out_shape=jax.ShapeDtypeStruct((_NH, _NB, _N), _bf),
    )(centers, qh)

    bias_tok = jnp.transpose(bias_bm, (0, 2, 1))

    out = pl.pallas_call(
        _attn_kernel,
        grid=(_N // _BQ, _NH),
        in_specs=[
            pl.BlockSpec((1, _BQ, _EH), lambda i, h: (h, i, 0)),
            pl.BlockSpec((1, _BQ, _NB), lambda i, h: (h, i, 0)),
            pl.BlockSpec((_NH, _N, _AUG), lambda i, h: (0, 0, 0)),
            pl.BlockSpec((_NH, _EH, _DIM), lambda i, h: (0, 0, 0)),
            pl.BlockSpec((1, _DIM), lambda i, h: (0, 0)),
        ],
        out_specs=pl.BlockSpec((_BQ, _DIM), lambda i, h: (i, 0)),
        out_shape=jax.ShapeDtypeStruct((_N, _DIM), _f32),
    )(qh, bias_tok, kb, Wp, bp)

    return out


# speedup vs baseline: 1.0616x; 1.0616x over previous
import jax
import jax.numpy as jnp
import numpy as np
from jax.experimental import pallas as pl

_DIM = 256
_NH = 8
_M = 128
_TOPK = 8
_DPOS = 3
_N = 4096
_NB = _N // _M
_EH = _DIM // _NH
_NEG = np.float32(-1e5)
_SCALE = np.float32(1.0 / np.sqrt(_EH))
_TS = 512
_BQ = 256
_PROJ_CHUNK = 512
_AUG = _EH + _NB
_bf = jnp.bfloat16
_f32 = jnp.float32


def _dot_t(a, b):
    return jax.lax.dot_general(a, b, (((1,), (1,)), ((), ())),
                               preferred_element_type=_f32)


def _qprep_kernel(x_ref, pos_ref, wpet_ref, bpe_ref, wq_ref, bq_ref,
                  wk_ref, bk_ref, q_ref, kbar_ref, kb_ref):
    i = pl.program_id(0)
    p = pos_ref[...]
    rel = (p - jnp.mean(p, axis=0, keepdims=True)).astype(_bf)
    pe = rel[:, 0:1].astype(_f32) * wpet_ref[0:1, :].astype(_f32)
    for d in range(1, _DPOS):
        pe = pe + rel[:, d:d + 1].astype(_f32) * wpet_ref[d:d + 1, :].astype(_f32)
    xp = (x_ref[...] + pe) + bpe_ref[...]
    xb = xp.astype(_bf)
    ind = (jax.lax.broadcasted_iota(jnp.int32, (_M, _NB), 1) == i).astype(_bf)
    for h in range(_NH):
        q = _dot_t(xb, wq_ref[h]) + bq_ref[h]
        q_ref[h, :, :] = q
        kb_ref[h, :, :] = jnp.concatenate([q.astype(_bf), ind], axis=1)
    k = _dot_t(xb, wk_ref[...]) + bk_ref[...]
    kbar_ref[0, :, :] = jnp.mean(k, axis=0, keepdims=True)


def _select_kernel(c_ref, q_ref, bias_ref):
    centers = c_ref[0].astype(_bf)
    q = q_ref[0].astype(_bf)
    v = _dot_t(centers, q)
    iota = jax.lax.broadcasted_iota(jnp.int32, (_NB, _TS), 0)
    sel = jnp.zeros((_NB, _TS), _f32)
    for _ in range(_TOPK):
        m = jnp.max(v, axis=0, keepdims=True)
        elig = v == m
        cand = jnp.where(elig, iota, np.int32(_NB * 2))
        bmin = jnp.min(cand, axis=0, keepdims=True)
        onehot = cand == bmin
        v = jnp.where(onehot, np.float32(-np.inf), v)
        sel = sel + onehot.astype(_f32)
    bias_ref[0, :, :] = jnp.where(sel > 0, np.float32(0.0), _NEG).astype(_bf)


def _attn_kernel(q_ref, bias_ref, kb_ref, o_ref):
    qa = jnp.concatenate(
        [(q_ref[0] * _SCALE).astype(_bf), bias_ref[0]], axis=1)
    kb = kb_ref[0]
    s = _dot_t(qa, kb)
    p = jnp.exp(s).astype(_bf)
    o = jax.lax.dot_general(
        p, kb, (((1,), (0,)), ((), ())), preferred_element_type=_f32)
    l = jnp.sum(o[:, _EH:], axis=1, keepdims=True)
    o_ref[0, :, :] = o[:, :_EH] / l


def _proj_kernel(a_ref, wp_ref, bp_ref, o_ref):
    acc = jnp.zeros((_PROJ_CHUNK, _DIM), _f32) + bp_ref[...]
    for h in range(_NH):
        acc = acc + jax.lax.dot_general(
            a_ref[h].astype(_bf), wp_ref[h], (((1,), (0,)), ((), ())),
            preferred_element_type=_f32)
    o_ref[...] = acc


def kernel(x, pos, W_qkv, b_qkv, W_proj, b_proj, W_pe, b_pe):
    Wq = W_qkv[0::3].reshape(_NH, _EH, _DIM).astype(_bf)
    bq = b_qkv[0::3].reshape(_NH, 1, _EH)
    Wk = W_qkv[1::3].astype(_bf)
    bk = b_qkv[1::3].reshape(1, _DIM)
    WpeT = W_pe.T.astype(_bf)
    bpe = b_pe.reshape(1, _DIM)
    Wp = W_proj.T.reshape(_NH, _EH, _DIM).astype(_bf)
    bp = b_proj.reshape(1, _DIM)

    qh, kbar, kb = pl.pallas_call(
        _qprep_kernel,
        grid=(_NB,),
        in_specs=[
            pl.BlockSpec((_M, _DIM), lambda i: (i, 0)),
            pl.BlockSpec((_M, _DPOS), lambda i: (i, 0)),
            pl.BlockSpec((_DPOS, _DIM), lambda i: (0, 0)),
            pl.BlockSpec((1, _DIM), lambda i: (0, 0)),
            pl.BlockSpec((_NH, _EH, _DIM), lambda i: (0, 0, 0)),
            pl.BlockSpec((_NH, 1, _EH), lambda i: (0, 0, 0)),
            pl.BlockSpec((_DIM, _DIM), lambda i: (0, 0)),
            pl.BlockSpec((1, _DIM), lambda i: (0, 0)),
        ],
        out_specs=[
            pl.BlockSpec((_NH, _M, _EH), lambda i: (0, i, 0)),
            pl.BlockSpec((1, 1, _DIM), lambda i: (i, 0, 0)),
            pl.BlockSpec((_NH, _M, _AUG), lambda i: (0, i, 0)),
        ],
        out_shape=[
            jax.ShapeDtypeStruct((_NH, _N, _EH), _f32),
            jax.ShapeDtypeStruct((_NB, 1, _DIM), _f32),
            jax.ShapeDtypeStruct((_NH, _N, _AUG), _bf),
        ],
    )(x, pos, WpeT, bpe, Wq, bq, Wk, bk)

    centers = jnp.transpose(kbar.reshape(_NB, _NH, _EH), (1, 0, 2))

    bias_bm = pl.pallas_call(
        _select_kernel,
        grid=(_NH, _N // _TS),
        in_specs=[
            pl.BlockSpec((1, _NB, _EH), lambda h, c: (h, 0, 0)),
            pl.BlockSpec((1, _TS, _EH), lambda h, c: (h, c, 0)),
        ],
        out_specs=pl.BlockSpec((1, _NB, _TS), lambda h, c: (h, 0, c)),
        out_shape=jax.ShapeDtypeStruct((_NH, _NB, _N), _bf),
    )(centers, qh)

    bias_tok = jnp.transpose(bias_bm, (0, 2, 1))

    attn = pl.pallas_call(
        _attn_kernel,
        grid=(_NH, _N // _BQ),
        in_specs=[
            pl.BlockSpec((1, _BQ, _EH), lambda h, i: (h, i, 0)),
            pl.BlockSpec((1, _BQ, _NB), lambda h, i: (h, i, 0)),
            pl.BlockSpec((1, _N, _AUG), lambda h, i: (h, 0, 0)),
        ],
        out_specs=pl.BlockSpec((1, _BQ, _EH), lambda h, i: (h, i, 0)),
        out_shape=jax.ShapeDtypeStruct((_NH, _N, _EH), _f32),
    )(qh, bias_tok, kb)

    out = pl.pallas_call(
        _proj_kernel,
        grid=(_N // _PROJ_CHUNK,),
        in_specs=[
            pl.BlockSpec((_NH, _PROJ_CHUNK, _EH), lambda r: (0, r, 0)),
            pl.BlockSpec((_NH, _EH, _DIM), lambda r: (0, 0, 0)),
            pl.BlockSpec((1, _DIM), lambda r: (0, 0)),
        ],
        out_specs=pl.BlockSpec((_PROJ_CHUNK, _DIM), lambda r: (r, 0)),
        out_shape=jax.ShapeDtypeStruct((_N, _DIM), _f32),
    )(attn, Wp, bp)

    return out


# speedup vs baseline: 1.0831x; 1.0203x over previous
import jax
import jax.numpy as jnp
import numpy as np
from jax.experimental import pallas as pl

_DIM = 256
_NH = 8
_M = 128
_TOPK = 8
_DPOS = 3
_N = 4096
_NB = _N // _M
_EH = _DIM // _NH
_NEG = np.float32(-1e5)
_SCALE = np.float32(1.0 / np.sqrt(_EH))
_TS = 512
_BQ = 256
_PROJ_CHUNK = 512
_AUG = _EH + _NB
_bf = jnp.bfloat16
_f32 = jnp.float32


def _dot_t(a, b):
    return jax.lax.dot_general(a, b, (((1,), (1,)), ((), ())),
                               preferred_element_type=_f32)


def _qprep_kernel(x_ref, pos_ref, wpet_ref, bpe_ref, wq_ref, bq_ref,
                  wk_ref, bk_ref, kbar_ref, kb_ref):
    i = pl.program_id(0)
    p = pos_ref[...]
    rel = (p - jnp.mean(p, axis=0, keepdims=True)).astype(_bf)
    pe = rel[:, 0:1].astype(_f32) * wpet_ref[0:1, :].astype(_f32)
    for d in range(1, _DPOS):
        pe = pe + rel[:, d:d + 1].astype(_f32) * wpet_ref[d:d + 1, :].astype(_f32)
    xp = (x_ref[...] + pe) + bpe_ref[...]
    xb = xp.astype(_bf)
    ind = (jax.lax.broadcasted_iota(jnp.int32, (_M, _NB), 1) == i).astype(_bf)
    for h in range(_NH):
        q = _dot_t(xb, wq_ref[h]) + bq_ref[h]
        kb_ref[h, :, :] = jnp.concatenate([q.astype(_bf), ind], axis=1)
    k = _dot_t(xb, wk_ref[...]) + bk_ref[...]
    kbar_ref[0, :, :] = jnp.mean(k, axis=0, keepdims=True)


def _select_kernel(c_ref, kb_ref, bias_ref):
    centers = c_ref[0].astype(_bf)
    q = kb_ref[0][:, :_EH]
    v = _dot_t(centers, q)
    iota = jax.lax.broadcasted_iota(jnp.int32, (_NB, _TS), 0)
    sel = jnp.zeros((_NB, _TS), _f32)
    for _ in range(_TOPK):
        m = jnp.max(v, axis=0, keepdims=True)
        elig = v == m
        cand = jnp.where(elig, iota, np.int32(_NB * 2))
        bmin = jnp.min(cand, axis=0, keepdims=True)
        onehot = cand == bmin
        v = jnp.where(onehot, np.float32(-np.inf), v)
        sel = sel + onehot.astype(_f32)
    bias_ref[0, :, :] = jnp.where(sel > 0, np.float32(0.0), _NEG).astype(_bf)


def _attn_kernel(bias_ref, kb_ref, o_ref):
    ib = pl.program_id(1)
    kb = kb_ref[0]
    qrows = kb_ref[0, pl.ds(ib * _BQ, _BQ), 0:_EH]
    qs = (qrows.astype(_f32) * _SCALE).astype(_bf)
    qa = jnp.concatenate([qs, bias_ref[0]], axis=1)
    s = _dot_t(qa, kb)
    p = jnp.exp(s).astype(_bf)
    o = jax.lax.dot_general(
        p, kb, (((1,), (0,)), ((), ())), preferred_element_type=_f32)
    l = jnp.sum(o[:, _EH:], axis=1, keepdims=True)
    o_ref[0, :, :] = o[:, :_EH] / l


def _proj_kernel(a_ref, wp_ref, bp_ref, o_ref):
    acc = jnp.zeros((_PROJ_CHUNK, _DIM), _f32) + bp_ref[...]
    for h in range(_NH):
        acc = acc + jax.lax.dot_general(
            a_ref[h].astype(_bf), wp_ref[h], (((1,), (0,)), ((), ())),
            preferred_element_type=_f32)
    o_ref[...] = acc


def kernel(x, pos, W_qkv, b_qkv, W_proj, b_proj, W_pe, b_pe):
    Wq = W_qkv[0::3].reshape(_NH, _EH, _DIM).astype(_bf)
    bq = b_qkv[0::3].reshape(_NH, 1, _EH)
    Wk = W_qkv[1::3].astype(_bf)
    bk = b_qkv[1::3].reshape(1, _DIM)
    WpeT = W_pe.T.astype(_bf)
    bpe = b_pe.reshape(1, _DIM)
    Wp = W_proj.T.reshape(_NH, _EH, _DIM).astype(_bf)
    bp = b_proj.reshape(1, _DIM)

    kbar, kb = pl.pallas_call(
        _qprep_kernel,
        grid=(_NB,),
        in_specs=[
            pl.BlockSpec((_M, _DIM), lambda i: (i, 0)),
            pl.BlockSpec((_M, _DPOS), lambda i: (i, 0)),
            pl.BlockSpec((_DPOS, _DIM), lambda i: (0, 0)),
            pl.BlockSpec((1, _DIM), lambda i: (0, 0)),
            pl.BlockSpec((_NH, _EH, _DIM), lambda i: (0, 0, 0)),
            pl.BlockSpec((_NH, 1, _EH), lambda i: (0, 0, 0)),
            pl.BlockSpec((_DIM, _DIM), lambda i: (0, 0)),
            pl.BlockSpec((1, _DIM), lambda i: (0, 0)),
        ],
        out_specs=[
            pl.BlockSpec((1, 1, _DIM), lambda i: (i, 0, 0)),
            pl.BlockSpec((_NH, _M, _AUG), lambda i: (0, i, 0)),
        ],
        out_shape=[
            jax.ShapeDtypeStruct((_NB, 1, _DIM), _f32),
            jax.ShapeDtypeStruct((_NH, _N, _AUG), _bf),
        ],
    )(x, pos, WpeT, bpe, Wq, bq, Wk, bk)

    centers = jnp.transpose(kbar.reshape(_NB, _NH, _EH), (1, 0, 2))

    bias_bm = pl.pallas_call(
        _select_kernel,
        grid=(_NH, _N // _TS),
        in_specs=[
            pl.BlockSpec((1, _NB, _EH), lambda h, c: (h, 0, 0)),
            pl.BlockSpec((1, _TS, _AUG), lambda h, c: (h, c, 0)),
        ],
        out_specs=pl.BlockSpec((1, _NB, _TS), lambda h, c: (h, 0, c)),
        out_shape=jax.ShapeDtypeStruct((_NH, _NB, _N), _bf),
    )(centers, kb)

    bias_tok = jnp.transpose(bias_bm, (0, 2, 1))

    attn = pl.pallas_call(
        _attn_kernel,
        grid=(_NH, _N // _BQ),
        in_specs=[
            pl.BlockSpec((1, _BQ, _NB), lambda h, i: (h, i, 0)),
            pl.BlockSpec((1, _N, _AUG), lambda h, i: (h, 0, 0)),
        ],
        out_specs=pl.BlockSpec((1, _BQ, _EH), lambda h, i: (h, i, 0)),
        out_shape=jax.ShapeDtypeStruct((_NH, _N, _EH), _f32),
    )(bias_tok, kb)

    out = pl.pallas_call(
        _proj_kernel,
        grid=(_N // _PROJ_CHUNK,),
        in_specs=[
            pl.BlockSpec((_NH, _PROJ_CHUNK, _EH), lambda r: (0, r, 0)),
            pl.BlockSpec((_NH, _EH, _DIM), lambda r: (0, 0, 0)),
            pl.BlockSpec((1, _DIM), lambda r: (0, 0)),
        ],
        out_specs=pl.BlockSpec((_PROJ_CHUNK, _DIM), lambda r: (r, 0)),
        out_shape=jax.ShapeDtypeStruct((_N, _DIM), _f32),
    )(attn, Wp, bp)

    return out
